# baseline (speedup 1.0000x reference)
import jax
import jax.numpy as jnp
from jax.experimental import pallas as pl
from jax.experimental.pallas import tpu as pltpu

N = 10000
BM = 400
GRID = N // BM


def _gcn(adj_ref, x_ref, W1_ref, b1_ref, W2_ref, b2_ref, out_ref,
         s_ref, g_ref):
    p = pl.program_id(0)
    i = pl.program_id(1)

    @pl.when((p == 0) & (i == 0))
    def _():
        s_ref[:] = jnp.dot(x_ref[:], W1_ref[:],
                           preferred_element_type=jnp.float32)

    adj_bf = adj_ref[:].astype(jnp.bfloat16)

    @pl.when(p == 0)
    def _():
        h = jnp.dot(adj_bf, s_ref[:].astype(jnp.bfloat16),
                    preferred_element_type=jnp.float32)
        h = jnp.maximum(h + b1_ref[:], 0.0)
        g_ref[pl.ds(i * BM, BM), :] = jnp.dot(
            h, W2_ref[:], preferred_element_type=jnp.float32)

    @pl.when(p == 1)
    def _():
        out_ref[:] = jnp.dot(adj_bf, g_ref[:].astype(jnp.bfloat16),
                             preferred_element_type=jnp.float32) + b2_ref[:]


def kernel(x, adj, W1, b1, W2, b2):
    nfeat = x.shape[1]
    nhid = W1.shape[1]
    nclass = W2.shape[1]
    b1_2d = b1.reshape(1, nhid)
    b2_2d = b2.reshape(1, nclass)

    full = lambda shape: pl.BlockSpec(shape, lambda p, i: (0, 0))

    out = pl.pallas_call(
        _gcn,
        grid=(2, GRID),
        in_specs=[
            pl.BlockSpec((BM, N), lambda p, i: (i, 0)),
            full((N, nfeat)),
            full((nfeat, nhid)),
            full((1, nhid)),
            full((nhid, nclass)),
            full((1, nclass)),
        ],
        out_specs=pl.BlockSpec((BM, nclass), lambda p, i: (p * i, 0)),
        out_shape=jax.ShapeDtypeStruct((N, nclass), jnp.float32),
        scratch_shapes=[
            pltpu.VMEM((N, nhid), jnp.float32),
            pltpu.VMEM((N, nclass), jnp.float32),
        ],
        compiler_params=pltpu.CompilerParams(
            dimension_semantics=("arbitrary", "arbitrary")),
    )(adj, x, W1, b1_2d, W2, b2_2d)

    return out


# speedup vs baseline: 1.0204x; 1.0204x over previous
import jax
import jax.numpy as jnp
from jax.experimental import pallas as pl
from jax.experimental.pallas import tpu as pltpu

N = 10000
BM = 512
GRID = -(-N // BM)
NPAD = GRID * BM

_CONTRACT_LAST = (((1,), (1,)), ((), ()))


def _gcn(adj_ref, xT_ref, W1T_ref, b1c_ref, W2T_ref, b2c_ref, outT_ref,
         sT_ref, gT_ref):
    p = pl.program_id(0)
    i = pl.program_id(1)

    @pl.when((p == 0) & (i == 0))
    def _():
        sT_ref[:] = jnp.dot(W1T_ref[:], xT_ref[:],
                            preferred_element_type=jnp.float32)

    @pl.when(p == 0)
    def _():
        hT = jax.lax.dot_general(sT_ref[:], adj_ref[:], _CONTRACT_LAST,
                                 preferred_element_type=jnp.float32)
        hT = jnp.maximum(hT + b1c_ref[:], 0.0)
        gT_ref[:, pl.ds(i * BM, BM)] = jnp.dot(
            W2T_ref[:], hT, preferred_element_type=jnp.float32)

    @pl.when(p == 1)
    def _():
        outT_ref[:, pl.ds(i * BM, BM)] = jax.lax.dot_general(
            gT_ref[:, :N], adj_ref[:], _CONTRACT_LAST,
            preferred_element_type=jnp.float32) + b2c_ref[:]


def kernel(x, adj, W1, b1, W2, b2):
    nfeat = x.shape[1]
    nhid = W1.shape[1]
    nclass = W2.shape[1]
    xT = x.T
    W1T = W1.T
    W2T = W2.T
    b1c = b1.reshape(nhid, 1)
    b2c = b2.reshape(nclass, 1)

    full = lambda shape: pl.BlockSpec(shape, lambda p, i: (0, 0))

    outT = pl.pallas_call(
        _gcn,
        grid=(2, GRID),
        in_specs=[
            pl.BlockSpec((BM, N), lambda p, i: (i, 0)),
            full((nfeat, N)),
            full((nhid, nfeat)),
            full((nhid, 1)),
            full((nclass, nhid)),
            full((nclass, 1)),
        ],
        out_specs=pl.BlockSpec((nclass, NPAD), lambda p, i: (0, 0)),
        out_shape=jax.ShapeDtypeStruct((nclass, NPAD), jnp.float32),
        scratch_shapes=[
            pltpu.VMEM((nhid, N), jnp.float32),
            pltpu.VMEM((nclass, NPAD), jnp.float32),
        ],
        compiler_params=pltpu.CompilerParams(
            dimension_semantics=("arbitrary", "arbitrary")),
    )(adj, xT, W1T, b1c, W2T, b2c)

    return outT[:, :N].T


# speedup vs baseline: 1.0373x; 1.0166x over previous
import jax
import jax.numpy as jnp
from jax.experimental import pallas as pl
from jax.experimental.pallas import tpu as pltpu

N = 10000
BM = 512
GRID = -(-N // BM)
NPAD = GRID * BM

_CONTRACT_LAST = (((1,), (1,)), ((), ()))
_CONTRACT_00 = (((0,), (0,)), ((), ()))


def _gcn(adj_ref, x_ref, W1_ref, b1c_ref, W2_ref, b2r_ref, out_ref,
         sT_ref, gT_ref):
    p = pl.program_id(0)
    i = pl.program_id(1)

    @pl.when((p == 0) & (i == 0))
    def _():
        s = jnp.dot(x_ref[:], W1_ref[:], preferred_element_type=jnp.float32)
        sT_ref[:] = s.T

    @pl.when(p == 0)
    def _():
        hT = jax.lax.dot_general(sT_ref[:], adj_ref[:], _CONTRACT_LAST,
                                 preferred_element_type=jnp.float32)
        hT = jnp.maximum(hT + b1c_ref[:], 0.0)
        gT_ref[:, pl.ds(i * BM, BM)] = jax.lax.dot_general(
            W2_ref[:], hT, _CONTRACT_00,
            preferred_element_type=jnp.float32)

    @pl.when(p == 1)
    def _():
        oT = jax.lax.dot_general(gT_ref[:, :N], adj_ref[:], _CONTRACT_LAST,
                                 preferred_element_type=jnp.float32)
        out_ref[:] = oT.T + b2r_ref[:]


def kernel(x, adj, W1, b1, W2, b2):
    nfeat = x.shape[1]
    nhid = W1.shape[1]
    nclass = W2.shape[1]
    b1c = b1.reshape(nhid, 1)
    b2r = b2.reshape(1, nclass)

    full = lambda shape: pl.BlockSpec(shape, lambda p, i: (0, 0))

    out = pl.pallas_call(
        _gcn,
        grid=(2, GRID),
        in_specs=[
            pl.BlockSpec((BM, N), lambda p, i: (i, 0)),
            full((N, nfeat)),
            full((nfeat, nhid)),
            full((nhid, 1)),
            full((nhid, nclass)),
            full((1, nclass)),
        ],
        out_specs=pl.BlockSpec((BM, nclass), lambda p, i: (p * i, 0)),
        out_shape=jax.ShapeDtypeStruct((N, nclass), jnp.float32),
        scratch_shapes=[
            pltpu.VMEM((nhid, N), jnp.float32),
            pltpu.VMEM((nclass, NPAD), jnp.float32),
        ],
        compiler_params=pltpu.CompilerParams(
            dimension_semantics=("arbitrary", "arbitrary")),
    )(adj, x, W1, b1c, W2, b2r)

    return out
